# Initial kernel scaffold; baseline (speedup 1.0000x reference)
import jax
import jax.numpy as jnp
from jax.experimental import pallas as pl


def kernel(nodes, history_va, history_af, feat_table, af_table, W1, b1):
    raise NotImplementedError("write your pallas kernel here")


# speedup vs baseline: 4.4951x; 4.4951x over previous
import functools

import jax
import jax.numpy as jnp
from jax import lax
from jax.experimental import pallas as pl
from jax.experimental.pallas import tpu as pltpu
from jax.experimental.pallas import tpu_sc as plsc

NC = 2
NS = 16
NW = NC * NS

CH = 8
DEG = 32
D = 128


@functools.lru_cache(maxsize=None)
def _build_sc_gather(b_pad, n_nodes):
    b_per_w = b_pad // NW
    n_chunks = b_per_w // CH
    idx_rows_per_w = (b_per_w * DEG) // D
    mesh = plsc.VectorSubcoreMesh(core_axis_name="c", subcore_axis_name="s")

    @functools.partial(
        pl.kernel,
        mesh=mesh,
        out_type=[
            jax.ShapeDtypeStruct((b_pad, D), jnp.float32),
            jax.ShapeDtypeStruct((b_pad, D), jnp.float32),
        ],
        scratch_types=[
            pltpu.VMEM((2, D), jnp.int32),
            pltpu.VMEM((CH,), jnp.int32),
            pltpu.VMEM((CH * DEG, D), jnp.float32),
            pltpu.VMEM((CH, D), jnp.float32),
            pltpu.VMEM((CH, D), jnp.float32),
            pltpu.SemaphoreType.DMA,
        ],
    )
    def sc_gather(hva_hbm, nodes_hbm, feat_hbm, vasum_hbm, self_hbm,
                  idx_v, sidx_v, rows_v, srows_v, out_v, sem):
        wid = lax.axis_index("s") * NC + lax.axis_index("c")
        node_base = wid * b_per_w
        irow_base = wid * idx_rows_per_w

        def chunk_body(c, carry):
            nb = node_base + c * CH
            ir = irow_base + c * 2
            pltpu.sync_copy(hva_hbm.at[pl.ds(ir, 2)], idx_v)
            pltpu.sync_copy(nodes_hbm.at[pl.ds(nb, CH)], sidx_v)
            cp0 = pltpu.make_async_copy(
                feat_hbm.at[idx_v.at[0]], rows_v.at[pl.ds(0, D)], sem)
            cp1 = pltpu.make_async_copy(
                feat_hbm.at[idx_v.at[1]], rows_v.at[pl.ds(D, D)], sem)
            cps = pltpu.make_async_copy(feat_hbm.at[sidx_v], srows_v, sem)
            cp0.start()
            cp1.start()
            cps.start()
            cp0.wait()
            cp1.wait()
            cps.wait()

            def node_body(i, carry2):
                r0 = i * DEG
                for j in range(D // 16):
                    sl = pl.ds(j * 16, 16)
                    acc = rows_v[r0, sl]
                    for r in range(1, DEG):
                        acc = acc + rows_v[r0 + r, sl]
                    out_v[i, sl] = acc
                return carry2

            lax.fori_loop(0, CH, node_body, 0)
            pltpu.sync_copy(out_v, vasum_hbm.at[pl.ds(nb, CH)])
            pltpu.sync_copy(srows_v, self_hbm.at[pl.ds(nb, CH)])
            return carry

        lax.fori_loop(0, n_chunks, chunk_body, 0)

    return sc_gather


def _tc_dense_body(self_ref, va_ref, af_ref, aft_ref, w_ref, b_ref, o_ref):
    bt = self_ref.shape[0]
    w = w_ref[...]
    w1a = w[:D, :]
    w1b = w[D:, :] * (1.0 / DEG)
    acc = jnp.dot(self_ref[...], w1a, preferred_element_type=jnp.float32)
    acc = acc + jnp.dot(va_ref[...], w1b, preferred_element_type=jnp.float32)
    p = jnp.dot(aft_ref[...], w1b, preferred_element_type=jnp.float32)
    af = af_ref[...]
    iota = lax.broadcasted_iota(jnp.int32, (bt, 64), 1)
    counts = jnp.zeros((bt, 64), jnp.float32)
    for k in range(DEG):
        counts = counts + (af[:, k][:, None] == iota).astype(jnp.float32)
    acc = acc + jnp.dot(counts, p, preferred_element_type=jnp.float32)
    o_ref[...] = jnp.maximum(acc + b_ref[...], 0.0)


@functools.lru_cache(maxsize=None)
def _build_tc_dense(b_pad, vocab):
    bt = 512
    grid = (b_pad // bt,)
    return pl.pallas_call(
        _tc_dense_body,
        grid=grid,
        in_specs=[
            pl.BlockSpec((bt, D), lambda i: (i, 0)),
            pl.BlockSpec((bt, D), lambda i: (i, 0)),
            pl.BlockSpec((bt, DEG), lambda i: (i, 0)),
            pl.BlockSpec((vocab, D), lambda i: (0, 0)),
            pl.BlockSpec((2 * D, D), lambda i: (0, 0)),
            pl.BlockSpec((1, D), lambda i: (0, 0)),
        ],
        out_specs=pl.BlockSpec((bt, D), lambda i: (i, 0)),
        out_shape=jax.ShapeDtypeStruct((b_pad, D), jnp.float32),
    )


def kernel(nodes, history_va, history_af, feat_table, af_table, W1, b1):
    b = nodes.shape[0]
    n_nodes = feat_table.shape[0]
    vocab = af_table.shape[0]
    b_pad = ((b + 8 * NW - 1) // (8 * NW)) * (8 * NW)
    pad = b_pad - b

    nodes = nodes.astype(jnp.int32)
    history_va = history_va.astype(jnp.int32)
    history_af = history_af.astype(jnp.int32)
    if pad:
        pad_nodes = jnp.arange(pad, dtype=jnp.int32) % n_nodes
        nodes_p = jnp.concatenate([nodes, pad_nodes])
        pad_h = (jnp.arange(pad * DEG, dtype=jnp.int32) % n_nodes).reshape(pad, DEG)
        hva_p = jnp.concatenate([history_va, pad_h], axis=0)
        haf_p = jnp.concatenate(
            [history_af, jnp.zeros((pad, DEG), jnp.int32)], axis=0)
    else:
        nodes_p, hva_p, haf_p = nodes, history_va, history_af
    hva_r = hva_p.reshape((b_pad * DEG) // D, D)

    va_sum, self_feats = _build_sc_gather(b_pad, n_nodes)(
        hva_r, nodes_p, feat_table)
    out = _build_tc_dense(b_pad, vocab)(
        self_feats, va_sum, haf_p, af_table, W1, b1.reshape(1, D))
    return out[:b]


# speedup vs baseline: 6.4745x; 1.4404x over previous
import functools

import jax
import jax.numpy as jnp
from jax import lax
from jax.experimental import pallas as pl
from jax.experimental.pallas import tpu as pltpu
from jax.experimental.pallas import tpu_sc as plsc

NC = 2
NS = 16
NW = NC * NS

CH = 8
DEG = 32
D = 128


@functools.lru_cache(maxsize=None)
def _build_sc_gather(b_pad, n_nodes):
    b_per_w = b_pad // NW
    n_chunks = b_per_w // CH
    idx_rows_per_w = (b_per_w * DEG) // D
    mesh = plsc.VectorSubcoreMesh(core_axis_name="c", subcore_axis_name="s")

    @functools.partial(
        pl.kernel,
        mesh=mesh,
        out_type=[
            jax.ShapeDtypeStruct((b_pad, D), jnp.float32),
            jax.ShapeDtypeStruct((b_pad, D), jnp.float32),
        ],
        scratch_types=[
            pltpu.VMEM((2, 2, D), jnp.int32),
            pltpu.VMEM((2, CH), jnp.int32),
            pltpu.VMEM((2, CH * DEG, D), jnp.float32),
            pltpu.VMEM((2, CH, D), jnp.float32),
            pltpu.VMEM((CH, D), jnp.float32),
            pltpu.SemaphoreType.DMA,
            pltpu.SemaphoreType.DMA,
        ],
    )
    def sc_gather(hva_hbm, nodes_hbm, feat_hbm, vasum_hbm, self_hbm,
                  idx_v, sidx_v, rows_v, srows_v, out_v, sem_a, sem_b):
        wid = lax.axis_index("s") * NC + lax.axis_index("c")
        node_base = wid * b_per_w
        irow_base = wid * idx_rows_per_w

        def start_chunk(c, buf, sem):
            nb = node_base + c * CH
            ir = irow_base + c * 2
            pltpu.sync_copy(hva_hbm.at[pl.ds(ir, 2)], idx_v.at[buf])
            pltpu.sync_copy(nodes_hbm.at[pl.ds(nb, CH)], sidx_v.at[buf])
            pltpu.make_async_copy(
                feat_hbm.at[idx_v.at[buf].at[0]],
                rows_v.at[buf].at[pl.ds(0, D)], sem).start()
            pltpu.make_async_copy(
                feat_hbm.at[idx_v.at[buf].at[1]],
                rows_v.at[buf].at[pl.ds(D, D)], sem).start()
            pltpu.make_async_copy(
                feat_hbm.at[sidx_v.at[buf]], srows_v.at[buf], sem).start()

        def finish_chunk(c, buf, sem):
            nb = node_base + c * CH
            pltpu.make_async_copy(
                feat_hbm.at[idx_v.at[buf].at[0]],
                rows_v.at[buf].at[pl.ds(0, D)], sem).wait()
            pltpu.make_async_copy(
                feat_hbm.at[idx_v.at[buf].at[1]],
                rows_v.at[buf].at[pl.ds(D, D)], sem).wait()
            pltpu.make_async_copy(
                feat_hbm.at[sidx_v.at[buf]], srows_v.at[buf], sem).wait()

            def node_body(i, carry2):
                r0 = i * DEG
                sls = [pl.ds(j * 16, 16) for j in range(D // 16)]
                accs = [rows_v[buf, r0, sl] for sl in sls]
                for r in range(1, DEG):
                    for j, sl in enumerate(sls):
                        accs[j] = accs[j] + rows_v[buf, r0 + r, sl]
                for j, sl in enumerate(sls):
                    out_v[i, sl] = accs[j]
                return carry2

            lax.fori_loop(0, CH, node_body, 0)
            pltpu.sync_copy(out_v, vasum_hbm.at[pl.ds(nb, CH)])
            pltpu.sync_copy(srows_v.at[buf], self_hbm.at[pl.ds(nb, CH)])

        start_chunk(0, 0, sem_a)

        def pair_body(p, carry):
            c0 = p * 2
            start_chunk(c0 + 1, 1, sem_b)
            finish_chunk(c0, 0, sem_a)

            @pl.when(p < (n_chunks // 2) - 1)
            def _():
                start_chunk(c0 + 2, 0, sem_a)

            finish_chunk(c0 + 1, 1, sem_b)
            return carry

        lax.fori_loop(0, n_chunks // 2, pair_body, 0)

    return sc_gather


def _tc_dense_body(self_ref, va_ref, af_ref, aft_ref, w_ref, b_ref, o_ref):
    bt = self_ref.shape[0]
    w = w_ref[...]
    w1a = w[:D, :]
    w1b = w[D:, :] * (1.0 / DEG)
    acc = jnp.dot(self_ref[...], w1a, preferred_element_type=jnp.float32)
    acc = acc + jnp.dot(va_ref[...], w1b, preferred_element_type=jnp.float32)
    p = jnp.dot(aft_ref[...], w1b, preferred_element_type=jnp.float32)
    af = af_ref[...]
    iota = lax.broadcasted_iota(jnp.int32, (bt, 64), 1)
    counts = jnp.zeros((bt, 64), jnp.float32)
    for k in range(DEG):
        counts = counts + (af[:, k][:, None] == iota).astype(jnp.float32)
    acc = acc + jnp.dot(counts, p, preferred_element_type=jnp.float32)
    o_ref[...] = jnp.maximum(acc + b_ref[...], 0.0)


@functools.lru_cache(maxsize=None)
def _build_tc_dense(b_pad, vocab):
    bt = 512
    grid = (b_pad // bt,)
    return pl.pallas_call(
        _tc_dense_body,
        grid=grid,
        in_specs=[
            pl.BlockSpec((bt, D), lambda i: (i, 0)),
            pl.BlockSpec((bt, D), lambda i: (i, 0)),
            pl.BlockSpec((bt, DEG), lambda i: (i, 0)),
            pl.BlockSpec((vocab, D), lambda i: (0, 0)),
            pl.BlockSpec((2 * D, D), lambda i: (0, 0)),
            pl.BlockSpec((1, D), lambda i: (0, 0)),
        ],
        out_specs=pl.BlockSpec((bt, D), lambda i: (i, 0)),
        out_shape=jax.ShapeDtypeStruct((b_pad, D), jnp.float32),
    )


def kernel(nodes, history_va, history_af, feat_table, af_table, W1, b1):
    b = nodes.shape[0]
    n_nodes = feat_table.shape[0]
    vocab = af_table.shape[0]
    b_pad = ((b + 8 * NW - 1) // (8 * NW)) * (8 * NW)
    pad = b_pad - b

    nodes = nodes.astype(jnp.int32)
    history_va = history_va.astype(jnp.int32)
    history_af = history_af.astype(jnp.int32)
    if pad:
        pad_nodes = jnp.arange(pad, dtype=jnp.int32) % n_nodes
        nodes_p = jnp.concatenate([nodes, pad_nodes])
        pad_h = (jnp.arange(pad * DEG, dtype=jnp.int32) % n_nodes).reshape(pad, DEG)
        hva_p = jnp.concatenate([history_va, pad_h], axis=0)
        haf_p = jnp.concatenate(
            [history_af, jnp.zeros((pad, DEG), jnp.int32)], axis=0)
    else:
        nodes_p, hva_p, haf_p = nodes, history_va, history_af
    hva_r = hva_p.reshape((b_pad * DEG) // D, D)

    va_sum, self_feats = _build_sc_gather(b_pad, n_nodes)(
        hva_r, nodes_p, feat_table)
    out = _build_tc_dense(b_pad, vocab)(
        self_feats, va_sum, haf_p, af_table, W1, b1.reshape(1, D))
    return out[:b]


# speedup vs baseline: 6.8425x; 1.0568x over previous
import functools

import jax
import jax.numpy as jnp
from jax import lax
from jax.experimental import pallas as pl
from jax.experimental.pallas import tpu as pltpu
from jax.experimental.pallas import tpu_sc as plsc

NC = 2
NS = 16
NW = NC * NS

CH = 8
DEG = 32
D = 128


@functools.lru_cache(maxsize=None)
def _build_sc_gather(b_pad, n_nodes):
    b_per_w = b_pad // NW
    b_per_sc = b_pad // NC
    n_chunks = b_per_w // CH
    idx_rows_per_w = (b_per_w * DEG) // D
    mesh = plsc.VectorSubcoreMesh(core_axis_name="c", subcore_axis_name="s")

    @functools.partial(
        pl.kernel,
        mesh=mesh,
        out_type=[
            jax.ShapeDtypeStruct((b_pad, D), jnp.float32),
            jax.ShapeDtypeStruct((b_pad, D), jnp.float32),
        ],
        scratch_types=[
            pltpu.VMEM((2, 2, D), jnp.int32),
            pltpu.VMEM((2, CH), jnp.int32),
            pltpu.VMEM((2, CH * DEG, D), jnp.float32),
            pltpu.VMEM((2, CH, D), jnp.float32),
            pltpu.VMEM((2, D), jnp.int32),
            pltpu.VMEM((CH, D), jnp.float32),
            pltpu.VMEM_SHARED((b_per_sc, D), jnp.float32),
            pltpu.SemaphoreType.DMA,
            pltpu.SemaphoreType.DMA,
        ],
    )
    def sc_gather(hva_hbm, nodes_hbm, feat_hbm, vasum_hbm, self_hbm,
                  idx_v, sidx_v, rows_v, srows_v, dst_v, zero_v, acc_s,
                  sem_a, sem_b):
        sid = lax.axis_index("s")
        wid = lax.axis_index("c") * NS + sid
        node_base = wid * b_per_w
        local_base = sid * b_per_w
        irow_base = wid * idx_rows_per_w

        def start_chunk(c, buf, sem):
            nb = node_base + c * CH
            ir = irow_base + c * 2
            pltpu.sync_copy(hva_hbm.at[pl.ds(ir, 2)], idx_v.at[buf])
            pltpu.sync_copy(nodes_hbm.at[pl.ds(nb, CH)], sidx_v.at[buf])
            pltpu.make_async_copy(
                feat_hbm.at[idx_v.at[buf].at[0]],
                rows_v.at[buf].at[pl.ds(0, D)], sem).start()
            pltpu.make_async_copy(
                feat_hbm.at[idx_v.at[buf].at[1]],
                rows_v.at[buf].at[pl.ds(D, D)], sem).start()
            pltpu.make_async_copy(
                feat_hbm.at[sidx_v.at[buf]], srows_v.at[buf], sem).start()

        def finish_chunk(c, buf, sem):
            nb = node_base + c * CH
            lb = local_base + c * CH
            pltpu.make_async_copy(
                feat_hbm.at[idx_v.at[buf].at[0]],
                rows_v.at[buf].at[pl.ds(0, D)], sem).wait()
            pltpu.make_async_copy(
                feat_hbm.at[idx_v.at[buf].at[1]],
                rows_v.at[buf].at[pl.ds(D, D)], sem).wait()
            pltpu.make_async_copy(
                feat_hbm.at[sidx_v.at[buf]], srows_v.at[buf], sem).wait()

            for g in range(16):
                dst_v[g // 8, pl.ds((g % 8) * 16, 16)] = jnp.full(
                    (16,), lb + g // 2, jnp.int32)
            pltpu.sync_copy(rows_v.at[buf].at[pl.ds(0, D)],
                            acc_s.at[dst_v.at[0]], add=True)
            pltpu.sync_copy(rows_v.at[buf].at[pl.ds(D, D)],
                            acc_s.at[dst_v.at[1]], add=True)
            pltpu.sync_copy(srows_v.at[buf], self_hbm.at[pl.ds(nb, CH)])

        for i in range(CH):
            for j in range(D // 16):
                zero_v[i, pl.ds(j * 16, 16)] = jnp.zeros((16,), jnp.float32)

        def zero_body(c, carry):
            pltpu.sync_copy(zero_v, acc_s.at[pl.ds(local_base + c * CH, CH)])
            return carry

        lax.fori_loop(0, n_chunks, zero_body, 0)

        start_chunk(0, 0, sem_a)

        def pair_body(p, carry):
            c0 = p * 2
            start_chunk(c0 + 1, 1, sem_b)
            finish_chunk(c0, 0, sem_a)

            @pl.when(p < (n_chunks // 2) - 1)
            def _():
                start_chunk(c0 + 2, 0, sem_a)

            finish_chunk(c0 + 1, 1, sem_b)
            return carry

        lax.fori_loop(0, n_chunks // 2, pair_body, 0)

        pltpu.sync_copy(acc_s.at[pl.ds(local_base, b_per_w)],
                        vasum_hbm.at[pl.ds(node_base, b_per_w)])

    return sc_gather


def _tc_dense_body(self_ref, va_ref, af_ref, aft_ref, w_ref, b_ref, o_ref):
    bt = self_ref.shape[0]
    w = w_ref[...]
    w1a = w[:D, :]
    w1b = w[D:, :] * (1.0 / DEG)
    acc = jnp.dot(self_ref[...], w1a, preferred_element_type=jnp.float32)
    acc = acc + jnp.dot(va_ref[...], w1b, preferred_element_type=jnp.float32)
    p = jnp.dot(aft_ref[...], w1b, preferred_element_type=jnp.float32)
    af = af_ref[...]
    iota = lax.broadcasted_iota(jnp.int32, (bt, 64), 1)
    counts = jnp.zeros((bt, 64), jnp.float32)
    for k in range(DEG):
        counts = counts + (af[:, k][:, None] == iota).astype(jnp.float32)
    acc = acc + jnp.dot(counts, p, preferred_element_type=jnp.float32)
    o_ref[...] = jnp.maximum(acc + b_ref[...], 0.0)


@functools.lru_cache(maxsize=None)
def _build_tc_dense(b_pad, vocab):
    bt = 512
    grid = (b_pad // bt,)
    return pl.pallas_call(
        _tc_dense_body,
        grid=grid,
        in_specs=[
            pl.BlockSpec((bt, D), lambda i: (i, 0)),
            pl.BlockSpec((bt, D), lambda i: (i, 0)),
            pl.BlockSpec((bt, DEG), lambda i: (i, 0)),
            pl.BlockSpec((vocab, D), lambda i: (0, 0)),
            pl.BlockSpec((2 * D, D), lambda i: (0, 0)),
            pl.BlockSpec((1, D), lambda i: (0, 0)),
        ],
        out_specs=pl.BlockSpec((bt, D), lambda i: (i, 0)),
        out_shape=jax.ShapeDtypeStruct((b_pad, D), jnp.float32),
    )


def kernel(nodes, history_va, history_af, feat_table, af_table, W1, b1):
    b = nodes.shape[0]
    n_nodes = feat_table.shape[0]
    vocab = af_table.shape[0]
    b_pad = ((b + 8 * NW - 1) // (8 * NW)) * (8 * NW)
    pad = b_pad - b

    nodes = nodes.astype(jnp.int32)
    history_va = history_va.astype(jnp.int32)
    history_af = history_af.astype(jnp.int32)
    if pad:
        pad_nodes = jnp.arange(pad, dtype=jnp.int32) % n_nodes
        nodes_p = jnp.concatenate([nodes, pad_nodes])
        pad_h = (jnp.arange(pad * DEG, dtype=jnp.int32) % n_nodes).reshape(pad, DEG)
        hva_p = jnp.concatenate([history_va, pad_h], axis=0)
        haf_p = jnp.concatenate(
            [history_af, jnp.zeros((pad, DEG), jnp.int32)], axis=0)
    else:
        nodes_p, hva_p, haf_p = nodes, history_va, history_af
    hva_r = hva_p.reshape((b_pad * DEG) // D, D)

    va_sum, self_feats = _build_sc_gather(b_pad, n_nodes)(
        hva_r, nodes_p, feat_table)
    out = _build_tc_dense(b_pad, vocab)(
        self_feats, va_sum, haf_p, af_table, W1, b1.reshape(1, D))
    return out[:b]


# speedup vs baseline: 8.5266x; 1.2461x over previous
import functools

import jax
import jax.numpy as jnp
from jax import lax
from jax.experimental import pallas as pl
from jax.experimental.pallas import tpu as pltpu
from jax.experimental.pallas import tpu_sc as plsc

NC = 2
NS = 16
NW = NC * NS

CH = 8
DEG = 32
D = 128


@functools.lru_cache(maxsize=None)
def _build_sc_gather(b_pad, n_nodes):
    b_per_w = b_pad // NW
    b_per_sc = b_pad // NC
    n_chunks = b_per_w // CH
    idx_rows_per_w = (b_per_w * DEG) // D
    zrows = 32
    mesh = plsc.VectorSubcoreMesh(core_axis_name="c", subcore_axis_name="s")

    @functools.partial(
        pl.kernel,
        mesh=mesh,
        out_type=[
            jax.ShapeDtypeStruct((b_pad, D), jnp.float32),
            jax.ShapeDtypeStruct((b_pad, D), jnp.float32),
        ],
        scratch_types=[
            pltpu.VMEM((idx_rows_per_w, D), jnp.int32),
            pltpu.VMEM((b_per_w,), jnp.int32),
            pltpu.VMEM((2, CH * DEG, D), jnp.float32),
            pltpu.VMEM((2, D), jnp.int32),
            pltpu.VMEM((zrows, D), jnp.float32),
            pltpu.VMEM_SHARED((b_per_sc, D), jnp.float32),
            pltpu.SemaphoreType.DMA,
            pltpu.SemaphoreType.DMA,
            pltpu.SemaphoreType.DMA,
        ],
    )
    def sc_gather(hva_hbm, nodes_hbm, feat_hbm, vasum_hbm, self_hbm,
                  idx_v, sidx_v, rows_v, dst_v, zero_v, acc_s,
                  sem_a, sem_b, sem_s):
        sid = lax.axis_index("s")
        wid = lax.axis_index("c") * NS + sid
        node_base = wid * b_per_w
        local_base = sid * b_per_w
        irow_base = wid * idx_rows_per_w

        pltpu.sync_copy(hva_hbm.at[pl.ds(irow_base, idx_rows_per_w)], idx_v)
        pltpu.sync_copy(nodes_hbm.at[pl.ds(node_base, b_per_w)], sidx_v)

        def start_chunk(c, buf, sem):
            ir = c * 2
            pltpu.make_async_copy(
                feat_hbm.at[idx_v.at[ir]],
                rows_v.at[buf].at[pl.ds(0, D)], sem).start()
            pltpu.make_async_copy(
                feat_hbm.at[idx_v.at[ir + 1]],
                rows_v.at[buf].at[pl.ds(D, D)], sem).start()

        def finish_chunk(c, buf, sem):
            lb = local_base + c * CH
            pltpu.make_async_copy(
                feat_hbm.at[idx_v.at[c * 2]],
                rows_v.at[buf].at[pl.ds(0, D)], sem).wait()
            pltpu.make_async_copy(
                feat_hbm.at[idx_v.at[c * 2 + 1]],
                rows_v.at[buf].at[pl.ds(D, D)], sem).wait()

            for g in range(16):
                dst_v[g // 8, pl.ds((g % 8) * 16, 16)] = jnp.full(
                    (16,), lb + g // 2, jnp.int32)
            pltpu.sync_copy(rows_v.at[buf].at[pl.ds(0, D)],
                            acc_s.at[dst_v.at[0]], add=True)
            pltpu.sync_copy(rows_v.at[buf].at[pl.ds(D, D)],
                            acc_s.at[dst_v.at[1]], add=True)

        for i in range(zrows):
            for j in range(D // 16):
                zero_v[i, pl.ds(j * 16, 16)] = jnp.zeros((16,), jnp.float32)

        def zero_body(c, carry):
            pltpu.sync_copy(
                zero_v, acc_s.at[pl.ds(local_base + c * zrows, zrows)])
            return carry

        lax.fori_loop(0, b_per_w // zrows, zero_body, 0)

        start_chunk(0, 0, sem_a)

        def pair_body(p, carry):
            c0 = p * 2
            start_chunk(c0 + 1, 1, sem_b)
            finish_chunk(c0, 0, sem_a)

            @pl.when(p < (n_chunks // 2) - 1)
            def _():
                start_chunk(c0 + 2, 0, sem_a)

            finish_chunk(c0 + 1, 1, sem_b)
            return carry

        lax.fori_loop(0, n_chunks // 2, pair_body, 0)

        self_cps = []
        off = 0
        while off < b_per_w:
            n = min(D, b_per_w - off)
            buf, roff = (off // (2 * D)) % 2, off % (2 * D)
            self_cps.append((off, n, pltpu.make_async_copy(
                feat_hbm.at[sidx_v.at[pl.ds(off, n)]],
                rows_v.at[buf].at[pl.ds(roff, n)], sem_s)))
            off += n
        for _, _, cp in self_cps:
            cp.start()
        pltpu.sync_copy(acc_s.at[pl.ds(local_base, b_per_w)],
                        vasum_hbm.at[pl.ds(node_base, b_per_w)])
        for off, n, cp in self_cps:
            cp.wait()
            buf, roff = (off // (2 * D)) % 2, off % (2 * D)
            pltpu.sync_copy(rows_v.at[buf].at[pl.ds(roff, n)],
                            self_hbm.at[pl.ds(node_base + off, n)])

    return sc_gather


def _tc_counts_body(af_ref, cnt_ref):
    bt = af_ref.shape[0]
    af = af_ref[...]
    iota = lax.broadcasted_iota(jnp.int32, (bt, 64), 1)
    counts = jnp.zeros((bt, 64), jnp.float32)
    for k in range(DEG):
        counts = counts + (af[:, k][:, None] == iota).astype(jnp.float32)
    cnt_ref[...] = counts


@functools.lru_cache(maxsize=None)
def _build_tc_counts(b_pad):
    bt = 512
    return pl.pallas_call(
        _tc_counts_body,
        grid=(b_pad // bt,),
        in_specs=[pl.BlockSpec((bt, DEG), lambda i: (i, 0))],
        out_specs=pl.BlockSpec((bt, 64), lambda i: (i, 0)),
        out_shape=jax.ShapeDtypeStruct((b_pad, 64), jnp.float32),
    )


def _tc_dense_body(self_ref, va_ref, cnt_ref, aft_ref, w_ref, b_ref, o_ref):
    w = w_ref[...]
    w1a = w[:D, :]
    w1b = w[D:, :] * (1.0 / DEG)
    acc = jnp.dot(self_ref[...], w1a, preferred_element_type=jnp.float32)
    acc = acc + jnp.dot(va_ref[...], w1b, preferred_element_type=jnp.float32)
    p = jnp.dot(aft_ref[...], w1b, preferred_element_type=jnp.float32)
    acc = acc + jnp.dot(cnt_ref[...], p, preferred_element_type=jnp.float32)
    o_ref[...] = jnp.maximum(acc + b_ref[...], 0.0)


@functools.lru_cache(maxsize=None)
def _build_tc_dense(b_pad, vocab):
    bt = 512
    return pl.pallas_call(
        _tc_dense_body,
        grid=(b_pad // bt,),
        in_specs=[
            pl.BlockSpec((bt, D), lambda i: (i, 0)),
            pl.BlockSpec((bt, D), lambda i: (i, 0)),
            pl.BlockSpec((bt, 64), lambda i: (i, 0)),
            pl.BlockSpec((vocab, D), lambda i: (0, 0)),
            pl.BlockSpec((2 * D, D), lambda i: (0, 0)),
            pl.BlockSpec((1, D), lambda i: (0, 0)),
        ],
        out_specs=pl.BlockSpec((bt, D), lambda i: (i, 0)),
        out_shape=jax.ShapeDtypeStruct((b_pad, D), jnp.float32),
    )


def kernel(nodes, history_va, history_af, feat_table, af_table, W1, b1):
    b = nodes.shape[0]
    n_nodes = feat_table.shape[0]
    vocab = af_table.shape[0]
    b_pad = ((b + 8 * NW - 1) // (8 * NW)) * (8 * NW)
    pad = b_pad - b

    nodes = nodes.astype(jnp.int32)
    history_va = history_va.astype(jnp.int32)
    history_af = history_af.astype(jnp.int32)
    if pad:
        pad_nodes = jnp.arange(pad, dtype=jnp.int32) % n_nodes
        nodes_p = jnp.concatenate([nodes, pad_nodes])
        pad_h = (jnp.arange(pad * DEG, dtype=jnp.int32) % n_nodes).reshape(pad, DEG)
        hva_p = jnp.concatenate([history_va, pad_h], axis=0)
        haf_p = jnp.concatenate(
            [history_af, jnp.zeros((pad, DEG), jnp.int32)], axis=0)
    else:
        nodes_p, hva_p, haf_p = nodes, history_va, history_af
    hva_r = hva_p.reshape((b_pad * DEG) // D, D)

    counts = _build_tc_counts(b_pad)(haf_p)
    va_sum, self_feats = _build_sc_gather(b_pad, n_nodes)(
        hva_r, nodes_p, feat_table)
    out = _build_tc_dense(b_pad, vocab)(
        self_feats, va_sum, counts, af_table, W1, b1.reshape(1, D))
    return out[:b]


# speedup vs baseline: 8.6353x; 1.0127x over previous
import functools

import jax
import jax.numpy as jnp
from jax import lax
from jax.experimental import pallas as pl
from jax.experimental.pallas import tpu as pltpu
from jax.experimental.pallas import tpu_sc as plsc

NC = 2
NS = 16
NW = NC * NS

CH = 8
DEG = 32
D = 128


@functools.lru_cache(maxsize=None)
def _build_sc_gather(b_pad, n_nodes):
    b_per_w = b_pad // NW
    b_per_sc = b_pad // NC
    n_chunks = b_per_w // CH
    idx_rows_per_w = (b_per_w * DEG) // D
    zrows = 32
    mesh = plsc.VectorSubcoreMesh(core_axis_name="c", subcore_axis_name="s")

    @functools.partial(
        pl.kernel,
        mesh=mesh,
        out_type=[
            jax.ShapeDtypeStruct((b_pad, D), jnp.float32),
            jax.ShapeDtypeStruct((b_pad, D), jnp.float32),
        ],
        scratch_types=[
            pltpu.VMEM((idx_rows_per_w, D), jnp.int32),
            pltpu.VMEM((b_per_w,), jnp.int32),
            pltpu.VMEM((2, CH * DEG, D), jnp.float32),
            pltpu.VMEM((2, D), jnp.int32),
            pltpu.VMEM((zrows, D), jnp.float32),
            pltpu.VMEM_SHARED((b_per_sc, D), jnp.float32),
            pltpu.SemaphoreType.DMA,
            pltpu.SemaphoreType.DMA,
            pltpu.SemaphoreType.DMA,
        ],
    )
    def sc_gather(hva_hbm, nodes_hbm, feat_hbm, vasum_hbm, self_hbm,
                  idx_v, sidx_v, rows_v, dst_v, zero_v, acc_s,
                  sem_a, sem_b, sem_s):
        sid = lax.axis_index("s")
        wid = lax.axis_index("c") * NS + sid
        node_base = wid * b_per_w
        local_base = sid * b_per_w
        irow_base = wid * idx_rows_per_w

        pltpu.sync_copy(hva_hbm.at[pl.ds(irow_base, idx_rows_per_w)], idx_v)
        pltpu.sync_copy(nodes_hbm.at[pl.ds(node_base, b_per_w)], sidx_v)

        def start_chunk(c, buf, sem):
            ir = c * 2
            pltpu.make_async_copy(
                feat_hbm.at[idx_v.at[ir]],
                rows_v.at[buf].at[pl.ds(0, D)], sem).start()
            pltpu.make_async_copy(
                feat_hbm.at[idx_v.at[ir + 1]],
                rows_v.at[buf].at[pl.ds(D, D)], sem).start()

        def finish_chunk(c, buf, sem):
            lb = local_base + c * CH
            pltpu.make_async_copy(
                feat_hbm.at[idx_v.at[c * 2]],
                rows_v.at[buf].at[pl.ds(0, D)], sem).wait()
            pltpu.make_async_copy(
                feat_hbm.at[idx_v.at[c * 2 + 1]],
                rows_v.at[buf].at[pl.ds(D, D)], sem).wait()

            for g in range(16):
                dst_v[g // 8, pl.ds((g % 8) * 16, 16)] = jnp.full(
                    (16,), lb + g // 2, jnp.int32)
            pltpu.sync_copy(rows_v.at[buf].at[pl.ds(0, D)],
                            acc_s.at[dst_v.at[0]], add=True)
            pltpu.sync_copy(rows_v.at[buf].at[pl.ds(D, D)],
                            acc_s.at[dst_v.at[1]], add=True)

        start_chunk(0, 0, sem_a)
        start_chunk(1, 1, sem_b)

        for i in range(zrows):
            for j in range(D // 16):
                zero_v[i, pl.ds(j * 16, 16)] = jnp.zeros((16,), jnp.float32)

        def zero_body(c, carry):
            pltpu.sync_copy(
                zero_v, acc_s.at[pl.ds(local_base + c * zrows, zrows)])
            return carry

        lax.fori_loop(0, b_per_w // zrows, zero_body, 0)

        def pair_body(p, carry):
            c0 = p * 2
            finish_chunk(c0, 0, sem_a)

            @pl.when(p < (n_chunks // 2) - 1)
            def _():
                start_chunk(c0 + 2, 0, sem_a)

            finish_chunk(c0 + 1, 1, sem_b)

            @pl.when(p < (n_chunks // 2) - 1)
            def _():
                start_chunk(c0 + 3, 1, sem_b)

            return carry

        lax.fori_loop(0, n_chunks // 2, pair_body, 0)

        self_cps = []
        off = 0
        while off < b_per_w:
            n = min(D, b_per_w - off)
            buf, roff = (off // (2 * D)) % 2, off % (2 * D)
            self_cps.append((off, n, pltpu.make_async_copy(
                feat_hbm.at[sidx_v.at[pl.ds(off, n)]],
                rows_v.at[buf].at[pl.ds(roff, n)], sem_s)))
            off += n
        for _, _, cp in self_cps:
            cp.start()
        pltpu.sync_copy(acc_s.at[pl.ds(local_base, b_per_w)],
                        vasum_hbm.at[pl.ds(node_base, b_per_w)])
        for off, n, cp in self_cps:
            cp.wait()
            buf, roff = (off // (2 * D)) % 2, off % (2 * D)
            pltpu.sync_copy(rows_v.at[buf].at[pl.ds(roff, n)],
                            self_hbm.at[pl.ds(node_base + off, n)])

    return sc_gather


def _tc_counts_body(af_ref, cnt_ref):
    bt = af_ref.shape[0]
    af = af_ref[...].astype(jnp.bfloat16)
    iota = lax.broadcasted_iota(jnp.int32, (bt, 64), 1).astype(jnp.bfloat16)
    counts = jnp.zeros((bt, 64), jnp.bfloat16)
    for k in range(DEG):
        counts = counts + (af[:, k][:, None] == iota).astype(jnp.bfloat16)
    cnt_ref[...] = counts.astype(jnp.float32)


@functools.lru_cache(maxsize=None)
def _build_tc_counts(b_pad):
    bt = 512
    return pl.pallas_call(
        _tc_counts_body,
        grid=(b_pad // bt,),
        in_specs=[pl.BlockSpec((bt, DEG), lambda i: (i, 0))],
        out_specs=pl.BlockSpec((bt, 64), lambda i: (i, 0)),
        out_shape=jax.ShapeDtypeStruct((b_pad, 64), jnp.float32),
    )


def _tc_dense_body(self_ref, va_ref, cnt_ref, aft_ref, w_ref, b_ref, o_ref):
    w = w_ref[...]
    w1a = w[:D, :]
    w1b = w[D:, :] * (1.0 / DEG)
    acc = jnp.dot(self_ref[...], w1a, preferred_element_type=jnp.float32)
    acc = acc + jnp.dot(va_ref[...], w1b, preferred_element_type=jnp.float32)
    p = jnp.dot(aft_ref[...], w1b, preferred_element_type=jnp.float32)
    acc = acc + jnp.dot(cnt_ref[...], p, preferred_element_type=jnp.float32)
    o_ref[...] = jnp.maximum(acc + b_ref[...], 0.0)


@functools.lru_cache(maxsize=None)
def _build_tc_dense(b_pad, vocab):
    bt = 512
    return pl.pallas_call(
        _tc_dense_body,
        grid=(b_pad // bt,),
        in_specs=[
            pl.BlockSpec((bt, D), lambda i: (i, 0)),
            pl.BlockSpec((bt, D), lambda i: (i, 0)),
            pl.BlockSpec((bt, 64), lambda i: (i, 0)),
            pl.BlockSpec((vocab, D), lambda i: (0, 0)),
            pl.BlockSpec((2 * D, D), lambda i: (0, 0)),
            pl.BlockSpec((1, D), lambda i: (0, 0)),
        ],
        out_specs=pl.BlockSpec((bt, D), lambda i: (i, 0)),
        out_shape=jax.ShapeDtypeStruct((b_pad, D), jnp.float32),
    )


def kernel(nodes, history_va, history_af, feat_table, af_table, W1, b1):
    b = nodes.shape[0]
    n_nodes = feat_table.shape[0]
    vocab = af_table.shape[0]
    b_pad = ((b + 8 * NW - 1) // (8 * NW)) * (8 * NW)
    pad = b_pad - b

    nodes = nodes.astype(jnp.int32)
    history_va = history_va.astype(jnp.int32)
    history_af = history_af.astype(jnp.int32)
    if pad:
        pad_nodes = jnp.arange(pad, dtype=jnp.int32) % n_nodes
        nodes_p = jnp.concatenate([nodes, pad_nodes])
        pad_h = (jnp.arange(pad * DEG, dtype=jnp.int32) % n_nodes).reshape(pad, DEG)
        hva_p = jnp.concatenate([history_va, pad_h], axis=0)
        haf_p = jnp.concatenate(
            [history_af, jnp.zeros((pad, DEG), jnp.int32)], axis=0)
    else:
        nodes_p, hva_p, haf_p = nodes, history_va, history_af
    hva_r = hva_p.reshape((b_pad * DEG) // D, D)

    va_sum, self_feats = _build_sc_gather(b_pad, n_nodes)(
        hva_r, nodes_p, feat_table)
    counts = _build_tc_counts(b_pad)(haf_p)
    out = _build_tc_dense(b_pad, vocab)(
        self_feats, va_sum, counts, af_table, W1, b1.reshape(1, D))
    return out[:b]


# speedup vs baseline: 8.8214x; 1.0215x over previous
import functools

import jax
import jax.numpy as jnp
from jax import lax
from jax.experimental import pallas as pl
from jax.experimental.pallas import tpu as pltpu
from jax.experimental.pallas import tpu_sc as plsc

NC = 2
NS = 16
NW = NC * NS

CH = 4
NBUF = 3
DEG = 32
D = 128


@functools.lru_cache(maxsize=None)
def _build_sc_gather(b_pad, n_nodes):
    b_per_w = b_pad // NW
    b_per_sc = b_pad // NC
    n_chunks = b_per_w // CH
    idx_rows_per_w = (b_per_w * DEG) // D
    zrows = 32
    mesh = plsc.VectorSubcoreMesh(core_axis_name="c", subcore_axis_name="s")

    @functools.partial(
        pl.kernel,
        mesh=mesh,
        out_type=[
            jax.ShapeDtypeStruct((b_pad, D), jnp.float32),
            jax.ShapeDtypeStruct((b_pad, D), jnp.float32),
        ],
        scratch_types=[
            pltpu.VMEM((idx_rows_per_w, D), jnp.int32),
            pltpu.VMEM((b_per_w,), jnp.int32),
            pltpu.VMEM((NBUF, CH * DEG, D), jnp.float32),
            pltpu.VMEM((NBUF, D), jnp.int32),
            pltpu.VMEM((zrows, D), jnp.float32),
            pltpu.VMEM_SHARED((b_per_sc, D), jnp.float32),
            pltpu.SemaphoreType.DMA,
            pltpu.SemaphoreType.DMA,
            pltpu.SemaphoreType.DMA,
            pltpu.SemaphoreType.DMA,
            pltpu.SemaphoreType.DMA,
            pltpu.SemaphoreType.DMA,
            pltpu.SemaphoreType.DMA,
        ],
    )
    def sc_gather(hva_hbm, nodes_hbm, feat_hbm, vasum_hbm, self_hbm,
                  idx_v, sidx_v, rows_v, dst_v, zero_v, acc_s,
                  sg0, sg1, sg2, ss0, ss1, ss2, sem_s):
        sem_g = [sg0, sg1, sg2]
        sem_sc = [ss0, ss1, ss2]
        sid = lax.axis_index("s")
        wid = lax.axis_index("c") * NS + sid
        node_base = wid * b_per_w
        local_base = sid * b_per_w
        irow_base = wid * idx_rows_per_w

        pltpu.sync_copy(hva_hbm.at[pl.ds(irow_base, idx_rows_per_w)], idx_v)
        pltpu.sync_copy(nodes_hbm.at[pl.ds(node_base, b_per_w)], sidx_v)

        def start_gather(c, buf):
            pltpu.make_async_copy(
                feat_hbm.at[idx_v.at[c]], rows_v.at[buf], sem_g[buf]).start()

        def wait_gather(c, buf):
            pltpu.make_async_copy(
                feat_hbm.at[idx_v.at[c]], rows_v.at[buf], sem_g[buf]).wait()

        def start_scatter(c, buf):
            lb = local_base + c * CH
            for g in range(8):
                dst_v[buf, pl.ds(g * 16, 16)] = jnp.full(
                    (16,), lb + g // 2, jnp.int32)
            pltpu.async_copy(rows_v.at[buf], acc_s.at[dst_v.at[buf]],
                             sem_sc[buf], add=True)

        def wait_scatter(buf):
            pltpu.make_async_copy(rows_v.at[buf], acc_s.at[dst_v.at[buf]],
                                  sem_sc[buf]).wait()

        for c in range(NBUF - 1):
            start_gather(c, c)

        for i in range(zrows):
            for j in range(D // 16):
                zero_v[i, pl.ds(j * 16, 16)] = jnp.zeros((16,), jnp.float32)

        def zero_body(c, carry):
            pltpu.sync_copy(
                zero_v, acc_s.at[pl.ds(local_base + c * zrows, zrows)])
            return carry

        lax.fori_loop(0, b_per_w // zrows, zero_body, 0)

        def ring_body(p, carry):
            c0 = p * NBUF
            for j in range(NBUF):
                c = c0 + j
                prv = (j + 2) % NBUF

                @pl.when(c < n_chunks)
                def _():
                    wait_gather(c, j)
                    start_scatter(c, j)

                @pl.when(jnp.logical_and(c >= 1, c - 1 < n_chunks))
                def _():
                    wait_scatter(prv)

                @pl.when(c + 2 < n_chunks)
                def _():
                    start_gather(c + 2, prv)

            return carry

        lax.fori_loop(0, (n_chunks + NBUF) // NBUF, ring_body, 0)

        self_cps = []
        off = 0
        while off < b_per_w:
            n = min(D, b_per_w - off)
            buf = off // D
            self_cps.append((off, n, buf, pltpu.make_async_copy(
                feat_hbm.at[sidx_v.at[pl.ds(off, n)]],
                rows_v.at[buf % NBUF].at[pl.ds(0, n)], sem_s)))
            off += n
        for _, _, _, cp in self_cps:
            cp.start()
        pltpu.sync_copy(acc_s.at[pl.ds(local_base, b_per_w)],
                        vasum_hbm.at[pl.ds(node_base, b_per_w)])
        for off, n, buf, cp in self_cps:
            cp.wait()
            pltpu.sync_copy(rows_v.at[buf % NBUF].at[pl.ds(0, n)],
                            self_hbm.at[pl.ds(node_base + off, n)])

    return sc_gather


def _tc_counts_body(af_ref, cnt_ref):
    bt = af_ref.shape[0]
    af = af_ref[...]
    iota = lax.broadcasted_iota(jnp.int32, (bt, 64), 1)
    counts = jnp.zeros((bt, 64), jnp.float32)
    for k in range(DEG):
        counts = counts + (af[:, k][:, None] == iota).astype(jnp.float32)
    cnt_ref[...] = counts


@functools.lru_cache(maxsize=None)
def _build_tc_counts(b_pad):
    bt = 512
    return pl.pallas_call(
        _tc_counts_body,
        grid=(b_pad // bt,),
        in_specs=[pl.BlockSpec((bt, DEG), lambda i: (i, 0))],
        out_specs=pl.BlockSpec((bt, 64), lambda i: (i, 0)),
        out_shape=jax.ShapeDtypeStruct((b_pad, 64), jnp.float32),
    )


def _tc_dense_body(self_ref, va_ref, cnt_ref, aft_ref, w_ref, b_ref, o_ref):
    w = w_ref[...]
    w1a = w[:D, :]
    w1b = w[D:, :] * (1.0 / DEG)
    acc = jnp.dot(self_ref[...], w1a, preferred_element_type=jnp.float32)
    acc = acc + jnp.dot(va_ref[...], w1b, preferred_element_type=jnp.float32)
    p = jnp.dot(aft_ref[...], w1b, preferred_element_type=jnp.float32)
    acc = acc + jnp.dot(cnt_ref[...], p, preferred_element_type=jnp.float32)
    o_ref[...] = jnp.maximum(acc + b_ref[...], 0.0)


@functools.lru_cache(maxsize=None)
def _build_tc_dense(b_pad, vocab):
    bt = 512
    return pl.pallas_call(
        _tc_dense_body,
        grid=(b_pad // bt,),
        in_specs=[
            pl.BlockSpec((bt, D), lambda i: (i, 0)),
            pl.BlockSpec((bt, D), lambda i: (i, 0)),
            pl.BlockSpec((bt, 64), lambda i: (i, 0)),
            pl.BlockSpec((vocab, D), lambda i: (0, 0)),
            pl.BlockSpec((2 * D, D), lambda i: (0, 0)),
            pl.BlockSpec((1, D), lambda i: (0, 0)),
        ],
        out_specs=pl.BlockSpec((bt, D), lambda i: (i, 0)),
        out_shape=jax.ShapeDtypeStruct((b_pad, D), jnp.float32),
    )


def kernel(nodes, history_va, history_af, feat_table, af_table, W1, b1):
    b = nodes.shape[0]
    n_nodes = feat_table.shape[0]
    vocab = af_table.shape[0]
    b_pad = ((b + 8 * NW - 1) // (8 * NW)) * (8 * NW)
    pad = b_pad - b

    nodes = nodes.astype(jnp.int32)
    history_va = history_va.astype(jnp.int32)
    history_af = history_af.astype(jnp.int32)
    if pad:
        pad_nodes = jnp.arange(pad, dtype=jnp.int32) % n_nodes
        nodes_p = jnp.concatenate([nodes, pad_nodes])
        pad_h = (jnp.arange(pad * DEG, dtype=jnp.int32) % n_nodes).reshape(pad, DEG)
        hva_p = jnp.concatenate([history_va, pad_h], axis=0)
        haf_p = jnp.concatenate(
            [history_af, jnp.zeros((pad, DEG), jnp.int32)], axis=0)
    else:
        nodes_p, hva_p, haf_p = nodes, history_va, history_af
    hva_r = hva_p.reshape((b_pad * DEG) // D, D)

    va_sum, self_feats = _build_sc_gather(b_pad, n_nodes)(
        hva_r, nodes_p, feat_table)
    counts = _build_tc_counts(b_pad)(haf_p)
    out = _build_tc_dense(b_pad, vocab)(
        self_feats, va_sum, counts, af_table, W1, b1.reshape(1, D))
    return out[:b]


# speedup vs baseline: 9.4510x; 1.0714x over previous
import functools

import jax
import jax.numpy as jnp
from jax import lax
from jax.experimental import pallas as pl
from jax.experimental.pallas import tpu as pltpu
from jax.experimental.pallas import tpu_sc as plsc

NC = 2
NS = 16
NW = NC * NS

CH = 4
NBUF = 3
DEG = 32
D = 128


@functools.lru_cache(maxsize=None)
def _build_sc_gather(b, b_pad, n_nodes):
    b_per_w = b_pad // NW
    b_per_sc = b_pad // NC
    n_chunks = b_per_w // CH
    idx_rows_per_w = (b_per_w * DEG) // D
    valid_last = b - (NW - 1) * b_per_w
    assert 0 < valid_last <= b_per_w and valid_last % 8 == 0
    assert valid_last <= D
    zrows = 32
    mesh = plsc.VectorSubcoreMesh(core_axis_name="c", subcore_axis_name="s")

    @functools.partial(
        pl.kernel,
        mesh=mesh,
        out_type=[
            jax.ShapeDtypeStruct((b, D), jnp.float32),
            jax.ShapeDtypeStruct((b, D), jnp.float32),
        ],
        scratch_types=[
            pltpu.VMEM((idx_rows_per_w, D), jnp.int32),
            pltpu.VMEM((b_per_w,), jnp.int32),
            pltpu.VMEM((NBUF, CH * DEG, D), jnp.float32),
            pltpu.VMEM((NBUF, D), jnp.int32),
            pltpu.VMEM((zrows, D), jnp.float32),
            pltpu.VMEM_SHARED((b_per_sc, D), jnp.float32),
            pltpu.SemaphoreType.DMA,
            pltpu.SemaphoreType.DMA,
            pltpu.SemaphoreType.DMA,
            pltpu.SemaphoreType.DMA,
            pltpu.SemaphoreType.DMA,
            pltpu.SemaphoreType.DMA,
            pltpu.SemaphoreType.DMA,
        ],
    )
    def sc_gather(hva_hbm, nodes_hbm, feat_hbm, vasum_hbm, self_hbm,
                  idx_v, sidx_v, rows_v, dst_v, zero_v, acc_s,
                  sg0, sg1, sg2, ss0, ss1, ss2, sem_s):
        sem_g = [sg0, sg1, sg2]
        sem_sc = [ss0, ss1, ss2]
        sid = lax.axis_index("s")
        wid = lax.axis_index("c") * NS + sid
        node_base = wid * b_per_w
        local_base = sid * b_per_w
        irow_base = wid * idx_rows_per_w

        pltpu.sync_copy(hva_hbm.at[pl.ds(irow_base, idx_rows_per_w)], idx_v)
        pltpu.sync_copy(nodes_hbm.at[pl.ds(node_base, b_per_w)], sidx_v)

        def start_gather(c, buf):
            pltpu.make_async_copy(
                feat_hbm.at[idx_v.at[c]], rows_v.at[buf], sem_g[buf]).start()

        def wait_gather(c, buf):
            pltpu.make_async_copy(
                feat_hbm.at[idx_v.at[c]], rows_v.at[buf], sem_g[buf]).wait()

        def start_scatter(c, buf):
            lb = local_base + c * CH
            for g in range(8):
                dst_v[buf, pl.ds(g * 16, 16)] = jnp.full(
                    (16,), lb + g // 2, jnp.int32)
            pltpu.async_copy(rows_v.at[buf], acc_s.at[dst_v.at[buf]],
                             sem_sc[buf], add=True)

        def wait_scatter(buf):
            pltpu.make_async_copy(rows_v.at[buf], acc_s.at[dst_v.at[buf]],
                                  sem_sc[buf]).wait()

        for c in range(NBUF - 1):
            start_gather(c, c)

        for i in range(zrows):
            for j in range(D // 16):
                zero_v[i, pl.ds(j * 16, 16)] = jnp.zeros((16,), jnp.float32)

        def zero_body(c, carry):
            pltpu.sync_copy(
                zero_v, acc_s.at[pl.ds(local_base + c * zrows, zrows)])
            return carry

        lax.fori_loop(0, b_per_w // zrows, zero_body, 0)

        def ring_body(p, carry):
            c0 = p * NBUF
            for j in range(NBUF):
                c = c0 + j
                prv = (j + 2) % NBUF

                @pl.when(c < n_chunks)
                def _():
                    wait_gather(c, j)
                    start_scatter(c, j)

                @pl.when(jnp.logical_and(c >= 1, c - 1 < n_chunks))
                def _():
                    wait_scatter(prv)

                @pl.when(c + 2 < n_chunks)
                def _():
                    start_gather(c + 2, prv)

            return carry

        lax.fori_loop(0, (n_chunks + NBUF) // NBUF, ring_body, 0)

        self_cps = []
        off = 0
        while off < b_per_w:
            n = min(D, b_per_w - off)
            buf = off // D
            self_cps.append((off, n, buf, pltpu.make_async_copy(
                feat_hbm.at[sidx_v.at[pl.ds(off, n)]],
                rows_v.at[buf % NBUF].at[pl.ds(0, n)], sem_s)))
            off += n
        for _, _, _, cp in self_cps:
            cp.start()

        last = NW - 1
        if valid_last == b_per_w:
            pltpu.sync_copy(acc_s.at[pl.ds(local_base, b_per_w)],
                            vasum_hbm.at[pl.ds(node_base, b_per_w)])
        else:
            @pl.when(wid < last)
            def _():
                pltpu.sync_copy(acc_s.at[pl.ds(local_base, b_per_w)],
                                vasum_hbm.at[pl.ds(node_base, b_per_w)])

            @pl.when(wid == last)
            def _():
                pltpu.sync_copy(acc_s.at[pl.ds(local_base, valid_last)],
                                vasum_hbm.at[pl.ds(node_base, valid_last)])

        for off, n, buf, cp in self_cps:
            cp.wait()
            if valid_last == b_per_w:
                pltpu.sync_copy(rows_v.at[buf % NBUF].at[pl.ds(0, n)],
                                self_hbm.at[pl.ds(node_base + off, n)])
                continue

            @pl.when(wid < last)
            def _():
                pltpu.sync_copy(rows_v.at[buf % NBUF].at[pl.ds(0, n)],
                                self_hbm.at[pl.ds(node_base + off, n)])

            if off == 0:
                @pl.when(wid == last)
                def _():
                    pltpu.sync_copy(
                        rows_v.at[buf % NBUF].at[pl.ds(0, valid_last)],
                        self_hbm.at[pl.ds(node_base, valid_last)])

    return sc_gather


def _pick_bt(b):
    for bt in (1024, 1000, 512, 500, 256, 200, 128, 8):
        if b % bt == 0 and bt % 8 == 0:
            return bt
    return 8


def _tc_counts_body(af_ref, cnt_ref):
    bt = af_ref.shape[0]
    af = af_ref[...]
    iota = lax.broadcasted_iota(jnp.int32, (bt, 64), 1)
    counts = jnp.zeros((bt, 64), jnp.float32)
    for k in range(DEG):
        counts = counts + (af[:, k][:, None] == iota).astype(jnp.float32)
    cnt_ref[...] = counts


@functools.lru_cache(maxsize=None)
def _build_tc_counts(b):
    bt = _pick_bt(b)
    return pl.pallas_call(
        _tc_counts_body,
        grid=(b // bt,),
        in_specs=[pl.BlockSpec((bt, DEG), lambda i: (i, 0))],
        out_specs=pl.BlockSpec((bt, 64), lambda i: (i, 0)),
        out_shape=jax.ShapeDtypeStruct((b, 64), jnp.float32),
    )


def _tc_dense_body(self_ref, va_ref, cnt_ref, aft_ref, w_ref, b_ref, o_ref):
    w = w_ref[...]
    w1a = w[:D, :]
    w1b = w[D:, :] * (1.0 / DEG)
    acc = jnp.dot(self_ref[...], w1a, preferred_element_type=jnp.float32)
    acc = acc + jnp.dot(va_ref[...], w1b, preferred_element_type=jnp.float32)
    p = jnp.dot(aft_ref[...], w1b, preferred_element_type=jnp.float32)
    acc = acc + jnp.dot(cnt_ref[...], p, preferred_element_type=jnp.float32)
    o_ref[...] = jnp.maximum(acc + b_ref[...], 0.0)


@functools.lru_cache(maxsize=None)
def _build_tc_dense(b, vocab):
    bt = _pick_bt(b)
    return pl.pallas_call(
        _tc_dense_body,
        grid=(b // bt,),
        in_specs=[
            pl.BlockSpec((bt, D), lambda i: (i, 0)),
            pl.BlockSpec((bt, D), lambda i: (i, 0)),
            pl.BlockSpec((bt, 64), lambda i: (i, 0)),
            pl.BlockSpec((vocab, D), lambda i: (0, 0)),
            pl.BlockSpec((2 * D, D), lambda i: (0, 0)),
            pl.BlockSpec((1, D), lambda i: (0, 0)),
        ],
        out_specs=pl.BlockSpec((bt, D), lambda i: (i, 0)),
        out_shape=jax.ShapeDtypeStruct((b, D), jnp.float32),
    )


def kernel(nodes, history_va, history_af, feat_table, af_table, W1, b1):
    b = nodes.shape[0]
    n_nodes = feat_table.shape[0]
    vocab = af_table.shape[0]
    b_pad = ((b + 8 * NW - 1) // (8 * NW)) * (8 * NW)
    pad = b_pad - b

    nodes = nodes.astype(jnp.int32)
    history_va = history_va.astype(jnp.int32)
    history_af = history_af.astype(jnp.int32)
    if pad:
        pad_nodes = jnp.arange(pad, dtype=jnp.int32) % n_nodes
        nodes_p = jnp.concatenate([nodes, pad_nodes])
        pad_h = (jnp.arange(pad * DEG, dtype=jnp.int32) % n_nodes).reshape(pad, DEG)
        hva_p = jnp.concatenate([history_va, pad_h], axis=0)
    else:
        nodes_p, hva_p = nodes, history_va
    hva_r = hva_p.reshape((b_pad * DEG) // D, D)

    va_sum, self_feats = _build_sc_gather(b, b_pad, n_nodes)(
        hva_r, nodes_p, feat_table)
    counts = _build_tc_counts(b)(history_af)
    return _build_tc_dense(b, vocab)(
        self_feats, va_sum, counts, af_table, W1, b1.reshape(1, D))


# speedup vs baseline: 9.6985x; 1.0262x over previous
import functools

import jax
import jax.numpy as jnp
from jax import lax
from jax.experimental import pallas as pl
from jax.experimental.pallas import tpu as pltpu
from jax.experimental.pallas import tpu_sc as plsc

NC = 2
NS = 16
NW = NC * NS

CH = 4
NBUF = 4
DEG = 32
D = 128


@functools.lru_cache(maxsize=None)
def _build_sc_gather(b, b_pad, n_nodes):
    b_per_w = b_pad // NW
    b_per_sc = b_pad // NC
    n_chunks = b_per_w // CH
    idx_rows_per_w = (b_per_w * DEG) // D
    valid_last = b - (NW - 1) * b_per_w
    assert 0 < valid_last <= b_per_w and valid_last % 8 == 0
    assert valid_last <= D
    zrows = 32
    mesh = plsc.VectorSubcoreMesh(core_axis_name="c", subcore_axis_name="s")

    @functools.partial(
        pl.kernel,
        mesh=mesh,
        out_type=[
            jax.ShapeDtypeStruct((b, D), jnp.float32),
            jax.ShapeDtypeStruct((b, D), jnp.float32),
        ],
        scratch_types=[
            pltpu.VMEM((idx_rows_per_w, D), jnp.int32),
            pltpu.VMEM((b_per_w,), jnp.int32),
            pltpu.VMEM((NBUF, CH * DEG, D), jnp.float32),
            pltpu.VMEM((NBUF, D), jnp.int32),
            pltpu.VMEM((zrows, D), jnp.float32),
            pltpu.VMEM_SHARED((b_per_sc, D), jnp.float32),
            pltpu.SemaphoreType.DMA,
            pltpu.SemaphoreType.DMA,
            pltpu.SemaphoreType.DMA,
            pltpu.SemaphoreType.DMA,
            pltpu.SemaphoreType.DMA,
            pltpu.SemaphoreType.DMA,
            pltpu.SemaphoreType.DMA,
            pltpu.SemaphoreType.DMA,
            pltpu.SemaphoreType.DMA,
        ],
    )
    def sc_gather(hva_hbm, nodes_hbm, feat_hbm, vasum_hbm, self_hbm,
                  idx_v, sidx_v, rows_v, dst_v, zero_v, acc_s,
                  sg0, sg1, sg2, sg3, ss0, ss1, ss2, ss3, sem_s):
        sem_g = [sg0, sg1, sg2, sg3]
        sem_sc = [ss0, ss1, ss2, ss3]
        sid = lax.axis_index("s")
        wid = lax.axis_index("c") * NS + sid
        node_base = wid * b_per_w
        local_base = sid * b_per_w
        irow_base = wid * idx_rows_per_w

        pltpu.sync_copy(hva_hbm.at[pl.ds(irow_base, idx_rows_per_w)], idx_v)
        pltpu.sync_copy(nodes_hbm.at[pl.ds(node_base, b_per_w)], sidx_v)

        def start_gather(c, buf):
            pltpu.make_async_copy(
                feat_hbm.at[idx_v.at[c]], rows_v.at[buf], sem_g[buf]).start()

        def wait_gather(c, buf):
            pltpu.make_async_copy(
                feat_hbm.at[idx_v.at[c]], rows_v.at[buf], sem_g[buf]).wait()

        def start_scatter(c, buf):
            lb = local_base + c * CH
            for g in range(8):
                dst_v[buf, pl.ds(g * 16, 16)] = jnp.full(
                    (16,), lb + g // 2, jnp.int32)
            pltpu.async_copy(rows_v.at[buf], acc_s.at[dst_v.at[buf]],
                             sem_sc[buf], add=True)

        def wait_scatter(buf):
            pltpu.make_async_copy(rows_v.at[buf], acc_s.at[dst_v.at[buf]],
                                  sem_sc[buf]).wait()

        for c in range(NBUF - 1):
            start_gather(c, c)

        for i in range(zrows):
            for j in range(D // 16):
                zero_v[i, pl.ds(j * 16, 16)] = jnp.zeros((16,), jnp.float32)

        def zero_body(c, carry):
            pltpu.sync_copy(
                zero_v, acc_s.at[pl.ds(local_base + c * zrows, zrows)])
            return carry

        lax.fori_loop(0, b_per_w // zrows, zero_body, 0)

        def ring_body(p, carry):
            c0 = p * NBUF
            for j in range(NBUF):
                c = c0 + j
                prv = (j + NBUF - 1) % NBUF

                @pl.when(c < n_chunks)
                def _():
                    wait_gather(c, j)
                    start_scatter(c, j)

                @pl.when(jnp.logical_and(c >= 1, c - 1 < n_chunks))
                def _():
                    wait_scatter(prv)

                @pl.when(c + (NBUF - 1) < n_chunks)
                def _():
                    start_gather(c + (NBUF - 1), prv)

            return carry

        lax.fori_loop(0, (n_chunks + NBUF) // NBUF, ring_body, 0)

        self_cps = []
        off = 0
        while off < b_per_w:
            n = min(D, b_per_w - off)
            buf = off // D
            self_cps.append((off, n, buf, pltpu.make_async_copy(
                feat_hbm.at[sidx_v.at[pl.ds(off, n)]],
                rows_v.at[buf % NBUF].at[pl.ds(0, n)], sem_s)))
            off += n
        for _, _, _, cp in self_cps:
            cp.start()

        last = NW - 1
        if valid_last == b_per_w:
            pltpu.sync_copy(acc_s.at[pl.ds(local_base, b_per_w)],
                            vasum_hbm.at[pl.ds(node_base, b_per_w)])
        else:
            @pl.when(wid < last)
            def _():
                pltpu.sync_copy(acc_s.at[pl.ds(local_base, b_per_w)],
                                vasum_hbm.at[pl.ds(node_base, b_per_w)])

            @pl.when(wid == last)
            def _():
                pltpu.sync_copy(acc_s.at[pl.ds(local_base, valid_last)],
                                vasum_hbm.at[pl.ds(node_base, valid_last)])

        for off, n, buf, cp in self_cps:
            cp.wait()
            if valid_last == b_per_w:
                pltpu.sync_copy(rows_v.at[buf % NBUF].at[pl.ds(0, n)],
                                self_hbm.at[pl.ds(node_base + off, n)])
                continue

            @pl.when(wid < last)
            def _():
                pltpu.sync_copy(rows_v.at[buf % NBUF].at[pl.ds(0, n)],
                                self_hbm.at[pl.ds(node_base + off, n)])

            if off == 0:
                @pl.when(wid == last)
                def _():
                    pltpu.sync_copy(
                        rows_v.at[buf % NBUF].at[pl.ds(0, valid_last)],
                        self_hbm.at[pl.ds(node_base, valid_last)])

    return sc_gather


def _pick_bt(b):
    for bt in (1024, 1000, 512, 500, 256, 200, 128, 8):
        if b % bt == 0 and bt % 8 == 0:
            return bt
    return 8


def _tc_counts_body(af_ref, cnt_ref):
    bt = af_ref.shape[0]
    af = af_ref[...]
    iota = lax.broadcasted_iota(jnp.int32, (bt, 64), 1)
    counts = jnp.zeros((bt, 64), jnp.float32)
    for k in range(DEG):
        counts = counts + (af[:, k][:, None] == iota).astype(jnp.float32)
    cnt_ref[...] = counts


@functools.lru_cache(maxsize=None)
def _build_tc_counts(b):
    bt = _pick_bt(b)
    return pl.pallas_call(
        _tc_counts_body,
        grid=(b // bt,),
        in_specs=[pl.BlockSpec((bt, DEG), lambda i: (i, 0))],
        out_specs=pl.BlockSpec((bt, 64), lambda i: (i, 0)),
        out_shape=jax.ShapeDtypeStruct((b, 64), jnp.float32),
    )


def _tc_dense_body(self_ref, va_ref, cnt_ref, aft_ref, w_ref, b_ref, o_ref):
    w = w_ref[...]
    w1a = w[:D, :]
    w1b = w[D:, :] * (1.0 / DEG)
    acc = jnp.dot(self_ref[...], w1a, preferred_element_type=jnp.float32)
    acc = acc + jnp.dot(va_ref[...], w1b, preferred_element_type=jnp.float32)
    p = jnp.dot(aft_ref[...], w1b, preferred_element_type=jnp.float32)
    acc = acc + jnp.dot(cnt_ref[...], p, preferred_element_type=jnp.float32)
    o_ref[...] = jnp.maximum(acc + b_ref[...], 0.0)


@functools.lru_cache(maxsize=None)
def _build_tc_dense(b, vocab):
    bt = _pick_bt(b)
    return pl.pallas_call(
        _tc_dense_body,
        grid=(b // bt,),
        in_specs=[
            pl.BlockSpec((bt, D), lambda i: (i, 0)),
            pl.BlockSpec((bt, D), lambda i: (i, 0)),
            pl.BlockSpec((bt, 64), lambda i: (i, 0)),
            pl.BlockSpec((vocab, D), lambda i: (0, 0)),
            pl.BlockSpec((2 * D, D), lambda i: (0, 0)),
            pl.BlockSpec((1, D), lambda i: (0, 0)),
        ],
        out_specs=pl.BlockSpec((bt, D), lambda i: (i, 0)),
        out_shape=jax.ShapeDtypeStruct((b, D), jnp.float32),
    )


def kernel(nodes, history_va, history_af, feat_table, af_table, W1, b1):
    b = nodes.shape[0]
    n_nodes = feat_table.shape[0]
    vocab = af_table.shape[0]
    b_pad = ((b + 8 * NW - 1) // (8 * NW)) * (8 * NW)
    pad = b_pad - b

    nodes = nodes.astype(jnp.int32)
    history_va = history_va.astype(jnp.int32)
    history_af = history_af.astype(jnp.int32)
    if pad:
        pad_nodes = jnp.arange(pad, dtype=jnp.int32) % n_nodes
        nodes_p = jnp.concatenate([nodes, pad_nodes])
        pad_h = (jnp.arange(pad * DEG, dtype=jnp.int32) % n_nodes).reshape(pad, DEG)
        hva_p = jnp.concatenate([history_va, pad_h], axis=0)
    else:
        nodes_p, hva_p = nodes, history_va
    hva_r = hva_p.reshape((b_pad * DEG) // D, D)

    va_sum, self_feats = _build_sc_gather(b, b_pad, n_nodes)(
        hva_r, nodes_p, feat_table)
    counts = _build_tc_counts(b)(history_af)
    return _build_tc_dense(b, vocab)(
        self_feats, va_sum, counts, af_table, W1, b1.reshape(1, D))


# speedup vs baseline: 9.7438x; 1.0047x over previous
import functools

import jax
import jax.numpy as jnp
from jax import lax
from jax.experimental import pallas as pl
from jax.experimental.pallas import tpu as pltpu
from jax.experimental.pallas import tpu_sc as plsc

NC = 2
NS = 16
NW = NC * NS

CH = 4
NBUF = 4
DEG = 32
D = 128


@functools.lru_cache(maxsize=None)
def _build_sc_gather(b, b_pad, n_nodes):
    b_per_w = b_pad // NW
    b_per_sc = b_pad // NC
    n_chunks = b_per_w // CH
    idx_rows_per_w = (b_per_w * DEG) // D
    valid_last = b - (NW - 1) * b_per_w
    assert 0 < valid_last <= b_per_w and valid_last % 8 == 0
    assert valid_last <= D
    zrows = 32
    mesh = plsc.VectorSubcoreMesh(core_axis_name="c", subcore_axis_name="s")

    @functools.partial(
        pl.kernel,
        mesh=mesh,
        out_type=[
            jax.ShapeDtypeStruct((b, D), jnp.float32),
            jax.ShapeDtypeStruct((b, D), jnp.float32),
        ],
        scratch_types=[
            pltpu.VMEM((idx_rows_per_w, D), jnp.int32),
            pltpu.VMEM((b_per_w,), jnp.int32),
            pltpu.VMEM((NBUF, CH * DEG, D), jnp.float32),
            pltpu.VMEM((NBUF, D), jnp.int32),
            pltpu.VMEM((zrows, D), jnp.float32),
            pltpu.VMEM_SHARED((b_per_sc, D), jnp.float32),
            pltpu.SemaphoreType.DMA,
            pltpu.SemaphoreType.DMA,
            pltpu.SemaphoreType.DMA,
            pltpu.SemaphoreType.DMA,
            pltpu.SemaphoreType.DMA,
            pltpu.SemaphoreType.DMA,
            pltpu.SemaphoreType.DMA,
            pltpu.SemaphoreType.DMA,
            pltpu.SemaphoreType.DMA,
        ],
    )
    def sc_gather(hva_hbm, nodes_hbm, feat_hbm, vasum_hbm, self_hbm,
                  idx_v, sidx_v, rows_v, dst_v, zero_v, acc_s,
                  sg0, sg1, sg2, sg3, ss0, ss1, ss2, ss3, sem_s):
        sem_g = [sg0, sg1, sg2, sg3]
        sem_sc = [ss0, ss1, ss2, ss3]
        sid = lax.axis_index("s")
        wid = lax.axis_index("c") * NS + sid
        node_base = wid * b_per_w
        local_base = sid * b_per_w
        irow_base = wid * idx_rows_per_w

        pltpu.sync_copy(hva_hbm.at[pl.ds(irow_base, idx_rows_per_w)], idx_v)
        pltpu.sync_copy(nodes_hbm.at[pl.ds(node_base, b_per_w)], sidx_v)

        def start_gather(c, buf):
            pltpu.make_async_copy(
                feat_hbm.at[idx_v.at[c]], rows_v.at[buf], sem_g[buf]).start()

        def wait_gather(c, buf):
            pltpu.make_async_copy(
                feat_hbm.at[idx_v.at[c]], rows_v.at[buf], sem_g[buf]).wait()

        def start_scatter(c, buf):
            lb = local_base + c * CH
            for g in range(8):
                dst_v[buf, pl.ds(g * 16, 16)] = jnp.full(
                    (16,), lb + g // 2, jnp.int32)
            pltpu.async_copy(rows_v.at[buf], acc_s.at[dst_v.at[buf]],
                             sem_sc[buf], add=True)

        def wait_scatter(buf):
            pltpu.make_async_copy(rows_v.at[buf], acc_s.at[dst_v.at[buf]],
                                  sem_sc[buf]).wait()

        for c in range(NBUF - 1):
            start_gather(c, c)

        for i in range(zrows):
            for j in range(D // 16):
                zero_v[i, pl.ds(j * 16, 16)] = jnp.zeros((16,), jnp.float32)

        def zero_body(c, carry):
            pltpu.sync_copy(
                zero_v, acc_s.at[pl.ds(local_base + c * zrows, zrows)])
            return carry

        lax.fori_loop(0, b_per_w // zrows, zero_body, 0)

        def ring_body(p, carry):
            c0 = p * NBUF
            for j in range(NBUF):
                c = c0 + j
                prv = (j + NBUF - 1) % NBUF

                @pl.when(c < n_chunks)
                def _():
                    wait_gather(c, j)
                    start_scatter(c, j)

                @pl.when(jnp.logical_and(c >= 1, c - 1 < n_chunks))
                def _():
                    wait_scatter(prv)

                @pl.when(c + (NBUF - 1) < n_chunks)
                def _():
                    start_gather(c + (NBUF - 1), prv)

            return carry

        lax.fori_loop(0, (n_chunks + NBUF) // NBUF, ring_body, 0)

        self_cps = []
        off = 0
        while off < b_per_w:
            n = min(D, b_per_w - off)
            buf = off // D
            sem = sem_g[buf % NBUF] if buf < NBUF else sem_sc[buf % NBUF]
            self_cps.append((off, n, buf, pltpu.make_async_copy(
                feat_hbm.at[sidx_v.at[pl.ds(off, n)]],
                rows_v.at[buf % NBUF].at[pl.ds(0, n)], sem)))
            off += n
        for _, _, _, cp in self_cps:
            cp.start()

        last = NW - 1
        if valid_last == b_per_w:
            pltpu.sync_copy(acc_s.at[pl.ds(local_base, b_per_w)],
                            vasum_hbm.at[pl.ds(node_base, b_per_w)])
        else:
            @pl.when(wid < last)
            def _():
                pltpu.sync_copy(acc_s.at[pl.ds(local_base, b_per_w)],
                                vasum_hbm.at[pl.ds(node_base, b_per_w)])

            @pl.when(wid == last)
            def _():
                pltpu.sync_copy(acc_s.at[pl.ds(local_base, valid_last)],
                                vasum_hbm.at[pl.ds(node_base, valid_last)])

        for off, n, buf, cp in self_cps:
            cp.wait()
            if valid_last == b_per_w:
                pltpu.sync_copy(rows_v.at[buf % NBUF].at[pl.ds(0, n)],
                                self_hbm.at[pl.ds(node_base + off, n)])
                continue

            @pl.when(wid < last)
            def _():
                pltpu.sync_copy(rows_v.at[buf % NBUF].at[pl.ds(0, n)],
                                self_hbm.at[pl.ds(node_base + off, n)])

            if off == 0:
                @pl.when(wid == last)
                def _():
                    pltpu.sync_copy(
                        rows_v.at[buf % NBUF].at[pl.ds(0, valid_last)],
                        self_hbm.at[pl.ds(node_base, valid_last)])

    return sc_gather


def _pick_bt(b):
    for bt in (1024, 1000, 512, 500, 256, 200, 128, 8):
        if b % bt == 0 and bt % 8 == 0:
            return bt
    return 8


def _tc_counts_body(af_ref, cnt_ref):
    bt = af_ref.shape[0]
    af = af_ref[...]
    iota = lax.broadcasted_iota(jnp.int32, (bt, 64), 1)
    counts = jnp.zeros((bt, 64), jnp.float32)
    for k in range(DEG):
        counts = counts + (af[:, k][:, None] == iota).astype(jnp.float32)
    cnt_ref[...] = counts


@functools.lru_cache(maxsize=None)
def _build_tc_counts(b):
    bt = _pick_bt(b)
    return pl.pallas_call(
        _tc_counts_body,
        grid=(b // bt,),
        in_specs=[pl.BlockSpec((bt, DEG), lambda i: (i, 0))],
        out_specs=pl.BlockSpec((bt, 64), lambda i: (i, 0)),
        out_shape=jax.ShapeDtypeStruct((b, 64), jnp.float32),
    )


def _tc_dense_body(self_ref, va_ref, cnt_ref, aft_ref, w_ref, b_ref, o_ref):
    w = w_ref[...]
    w1a = w[:D, :]
    w1b = w[D:, :] * (1.0 / DEG)
    acc = jnp.dot(self_ref[...], w1a, preferred_element_type=jnp.float32)
    acc = acc + jnp.dot(va_ref[...], w1b, preferred_element_type=jnp.float32)
    p = jnp.dot(aft_ref[...], w1b, preferred_element_type=jnp.float32)
    acc = acc + jnp.dot(cnt_ref[...], p, preferred_element_type=jnp.float32)
    o_ref[...] = jnp.maximum(acc + b_ref[...], 0.0)


@functools.lru_cache(maxsize=None)
def _build_tc_dense(b, vocab):
    bt = _pick_bt(b)
    return pl.pallas_call(
        _tc_dense_body,
        grid=(b // bt,),
        in_specs=[
            pl.BlockSpec((bt, D), lambda i: (i, 0)),
            pl.BlockSpec((bt, D), lambda i: (i, 0)),
            pl.BlockSpec((bt, 64), lambda i: (i, 0)),
            pl.BlockSpec((vocab, D), lambda i: (0, 0)),
            pl.BlockSpec((2 * D, D), lambda i: (0, 0)),
            pl.BlockSpec((1, D), lambda i: (0, 0)),
        ],
        out_specs=pl.BlockSpec((bt, D), lambda i: (i, 0)),
        out_shape=jax.ShapeDtypeStruct((b, D), jnp.float32),
    )


def kernel(nodes, history_va, history_af, feat_table, af_table, W1, b1):
    b = nodes.shape[0]
    n_nodes = feat_table.shape[0]
    vocab = af_table.shape[0]
    b_pad = ((b + 8 * NW - 1) // (8 * NW)) * (8 * NW)
    pad = b_pad - b

    nodes = nodes.astype(jnp.int32)
    history_va = history_va.astype(jnp.int32)
    history_af = history_af.astype(jnp.int32)
    if pad:
        pad_nodes = jnp.arange(pad, dtype=jnp.int32) % n_nodes
        nodes_p = jnp.concatenate([nodes, pad_nodes])
        pad_h = (jnp.arange(pad * DEG, dtype=jnp.int32) % n_nodes).reshape(pad, DEG)
        hva_p = jnp.concatenate([history_va, pad_h], axis=0)
    else:
        nodes_p, hva_p = nodes, history_va
    hva_r = hva_p.reshape((b_pad * DEG) // D, D)

    va_sum, self_feats = _build_sc_gather(b, b_pad, n_nodes)(
        hva_r, nodes_p, feat_table)
    counts = _build_tc_counts(b)(history_af)
    return _build_tc_dense(b, vocab)(
        self_feats, va_sum, counts, af_table, W1, b1.reshape(1, D))
